# baseline (speedup 1.0000x reference)
import functools

import jax
import jax.numpy as jnp
from jax import lax
from jax.experimental import pallas as pl
from jax.experimental.pallas import tpu as pltpu
from jax.experimental.pallas import tpu_sc as plsc

NUM_EMB = 1000000
DIM = 64
BATCH = 16384
FIELDS = 26
NW = 32
TCOLS = 7813
NPAIR = TCOLS * 64
K1_SLOTS = 245


def _mesh():
    return plsc.VectorSubcoreMesh(core_axis_name="c", subcore_axis_name="s")


def _make_transpose_kernel():
    @functools.partial(
        pl.kernel,
        mesh=_mesh(),
        out_type=jax.ShapeDtypeStruct((NPAIR, 128), jnp.float32),
        compiler_params=pltpu.CompilerParams(use_tc_tiling_on_sc=True, needs_layout_passes=False),
        scratch_types=[
            pltpu.VMEM((64, 128), jnp.float32),
            pltpu.VMEM((64, 128), jnp.float32),
            pltpu.VMEM((64, 64), jnp.float32),
        ],
    )
    def k1(wT, out, in_b, out_b, in_tail):
        tid = lax.axis_index("s") * 2 + lax.axis_index("c")
        iota = lax.iota(jnp.int32, 16)

        def transpose_unit(in_ref, out_ref, n_rows):
            def krow(kl, carry):
                for p in range(2):
                    rvec = jnp.full((16,), 2 * kl + p, jnp.int32)
                    for j in range(4):
                        dvec = iota + (16 * j)
                        vals = plsc.load_gather(in_ref, [dvec, rvec])
                        out_ref[kl, pl.ds(64 * p + 16 * j, 16)] = vals
                return carry

            lax.fori_loop(0, n_rows, krow, 0)

        def body(cl, carry):
            c = cl * NW + tid

            @pl.when(c < TCOLS - 1)
            def _():
                pltpu.sync_copy(wT.at[:, pl.ds(c * 128, 128)], in_b)
                transpose_unit(in_b, out_b, 64)
                pltpu.sync_copy(out_b, out.at[pl.ds(c * 64, 64), :])

            return carry

        lax.fori_loop(0, K1_SLOTS, body, 0)

        @pl.when(tid == (TCOLS - 1) % NW)
        def _():
            pltpu.sync_copy(wT.at[:, pl.ds(999936, 64)], in_tail)
            transpose_unit(in_tail, out_b, 32)
            pltpu.sync_copy(out_b.at[pl.ds(0, 32), :],
                            out.at[pl.ds((TCOLS - 1) * 64, 32), :])

    return k1


def _make_gather_kernel():
    @functools.partial(
        pl.kernel,
        mesh=_mesh(),
        out_type=jax.ShapeDtypeStruct((FIELDS, DIM, BATCH), jnp.float32),
        compiler_params=pltpu.CompilerParams(use_tc_tiling_on_sc=True, needs_layout_passes=False),
        scratch_types=[
            pltpu.VMEM((32, 128), jnp.int32),
            pltpu.VMEM((1, 128), jnp.int32),
            pltpu.VMEM((1, 128), jnp.int32),
            pltpu.VMEM((128, 128), jnp.float32),
            pltpu.VMEM((64, 128), jnp.float32),
            pltpu.SemaphoreType.DMA,
        ],
    )
    def k2(idxP, table, out, idxv, kb, pb, rows, stg, gsem):
        tid = lax.axis_index("s") * 2 + lax.axis_index("c")
        iota = lax.iota(jnp.int32, 16)

        def do_field(f, c_b):
            for m in range(8):
                v = idxv[f, pl.ds(16 * m, 16)]
                v = jnp.minimum(jnp.maximum(v, 0), NUM_EMB - 1)
                kb[0, pl.ds(16 * m, 16)] = jnp.right_shift(v, 1)
                pb[0, pl.ds(16 * m, 16)] = jnp.left_shift(
                    jnp.bitwise_and(v, 1), 6)
            pltpu.async_copy(table.at[kb.at[0]], rows, gsem).wait()
            def drow(d, carry):
                for m in range(8):
                    bvec = iota + (16 * m)
                    colv = pb[0, pl.ds(16 * m, 16)] + d
                    stg[d, pl.ds(16 * m, 16)] = plsc.load_gather(
                        rows, [bvec, colv])
                return carry

            lax.fori_loop(0, DIM, drow, 0)
            pltpu.sync_copy(stg, out.at[f, :, pl.ds(c_b * 128, 128)])

        def cb_body(cl, carry):
            c_b = tid * 4 + cl
            for g in range(4):
                pltpu.sync_copy(
                    idxP.at[pl.ds(8 * g, 8), pl.ds(c_b * 128, 128)],
                    idxv.at[pl.ds(8 * g, 8), :])

            def f_body(f, carry2):
                do_field(f, c_b)
                return carry2

            lax.fori_loop(0, FIELDS, f_body, 0)
            return carry

        lax.fori_loop(0, 4, cb_body, 0)

    return k2


_k1 = _make_transpose_kernel()
_k2 = _make_gather_kernel()


@jax.jit
def kernel(input_indices, weight):
    wT = weight.T
    table = _k1(wT)
    idxP = jnp.pad(input_indices.T, ((0, 6), (0, 0)))
    outT = _k2(idxP, table)
    return jnp.transpose(outT, (2, 0, 1))


# speedup vs baseline: 3.1673x; 3.1673x over previous
import functools

import jax
import jax.numpy as jnp
from jax import lax
from jax.experimental import pallas as pl
from jax.experimental.pallas import tpu as pltpu
from jax.experimental.pallas import tpu_sc as plsc

NUM_EMBEDDINGS = 1000000
EMBEDDING_DIM = 64
BATCH = 16384
FIELDS = 26

B_TOTAL = BATCH * FIELDS
NW = 32
B_PER_W = B_TOTAL // NW
CHUNK = 832
N_CHUNKS = B_PER_W // CHUNK


def _make_gather_kernel():
    mesh = plsc.VectorSubcoreMesh(core_axis_name="c", subcore_axis_name="s")

    @functools.partial(
        pl.kernel,
        mesh=mesh,
        out_type=jax.ShapeDtypeStruct((B_TOTAL, EMBEDDING_DIM), jnp.float32),
        compiler_params=pltpu.CompilerParams(use_tc_tiling_on_sc=False),
        scratch_types=[
            pltpu.VMEM((CHUNK,), jnp.int32),
            pltpu.VMEM((CHUNK,), jnp.int32),
            pltpu.VMEM((CHUNK, EMBEDDING_DIM), jnp.float32),
            pltpu.VMEM((CHUNK, EMBEDDING_DIM), jnp.float32),
            pltpu.SemaphoreType.DMA,
            pltpu.SemaphoreType.DMA,
            pltpu.SemaphoreType.DMA,
            pltpu.SemaphoreType.DMA,
        ],
    )
    def gather_kernel(table_hbm, idx_hbm, out_hbm,
                      idx0, idx1, rows0, rows1,
                      gsem0, gsem1, wsem0, wsem1):
        wid = lax.axis_index("s") * 2 + lax.axis_index("c")
        w_base = wid * B_PER_W

        idx_v = (idx0, idx1)
        rows_v = (rows0, rows1)
        gsem = (gsem0, gsem1)
        wsem = (wsem0, wsem1)
        gathers = [None, None]
        writes = [None, None]

        for i in range(N_CHUNKS):
            b = i % 2
            base = w_base + i * CHUNK
            if writes[b] is not None:
                writes[b].wait()
            pltpu.sync_copy(idx_hbm.at[pl.ds(base, CHUNK)], idx_v[b])
            gathers[b] = pltpu.async_copy(table_hbm.at[idx_v[b]], rows_v[b], gsem[b])
            if i >= 1:
                pb = (i - 1) % 2
                pbase = w_base + (i - 1) * CHUNK
                gathers[pb].wait()
                writes[pb] = pltpu.async_copy(
                    rows_v[pb], out_hbm.at[pl.ds(pbase, CHUNK)], wsem[pb])

        last = N_CHUNKS - 1
        lb = last % 2
        gathers[lb].wait()
        writes[lb] = pltpu.async_copy(
            rows_v[lb], out_hbm.at[pl.ds(w_base + last * CHUNK, CHUNK)], wsem[lb])
        writes[0].wait()
        writes[1].wait()

    return gather_kernel


_gather = _make_gather_kernel()


@jax.jit
def kernel(input_indices, weight):
    idx_flat = input_indices.reshape(B_TOTAL)
    out_flat = _gather(weight, idx_flat)
    return out_flat.reshape(BATCH, FIELDS, EMBEDDING_DIM)
